# Initial kernel scaffold; baseline (speedup 1.0000x reference)
import jax
import jax.numpy as jnp
from jax.experimental import pallas as pl


def kernel(x, edge_index, edge_weight, W1, b1, W2, b2):
    raise NotImplementedError("write your pallas kernel here")


# speedup vs baseline: 2.7857x; 2.7857x over previous
import functools

import jax
import jax.numpy as jnp
from jax import lax
from jax.experimental import pallas as pl
from jax.experimental.pallas import tpu as pltpu
from jax.experimental.pallas import tpu_sc as plsc

N_PAD = 10240
CHUNK = 128
LANES = 16
N_SUBCORES = 16
N_CORES = 2
ROWS_PER_TILE = N_PAD // N_SUBCORES
DRAIN_CHUNK = 128
MM_BLOCK = 1024


def _zero_rows(rows_ref, nrows, ncols):
    zv = jnp.zeros((LANES,), jnp.float32)

    def body(i, _):
        for k in range(ncols // LANES):
            rows_ref[i, pl.ds(k * LANES, LANES)] = zv
        return 0

    lax.fori_loop(0, nrows, body, 0)


def _spmm_accumulate(h_hbm, acc, srcbuf, dstbuf, wbuf, rows, gsem, nch, ncols):
    def chunk_body(j, _):
        pltpu.async_copy(h_hbm.at[srcbuf.at[j]], rows, gsem).wait()

        def edge_group(g, _):
            base = g * LANES
            wvec = wbuf[j, pl.ds(g * LANES, LANES)]
            for i in range(LANES):
                wv = wvec[i]
                for k in range(ncols // LANES):
                    sl = pl.ds(k * LANES, LANES)
                    rows[base + i, sl] = rows[base + i, sl] * wv
            return 0

        lax.fori_loop(0, CHUNK // LANES, edge_group, 0)

        pltpu.sync_copy(rows, acc.at[dstbuf.at[j]], add=True)
        return 0

    lax.fori_loop(0, nch, chunk_body, 0)


def _drain(acc, dbuf, out_hbm, row0):
    for j in range(ROWS_PER_TILE // DRAIN_CHUNK):
        r = row0 + j * DRAIN_CHUNK
        pltpu.sync_copy(acc.at[pl.ds(r, DRAIN_CHUNK)], dbuf)
        pltpu.sync_copy(dbuf, out_hbm.at[pl.ds(r, DRAIN_CHUNK)])


def _make_spmm1(n, nch, ncols):
    mesh = plsc.VectorSubcoreMesh(core_axis_name="c", subcore_axis_name="s")

    @functools.partial(
        pl.kernel,
        mesh=mesh,
        out_type=[
            jax.ShapeDtypeStruct((n, ncols), jnp.float32),
            jax.ShapeDtypeStruct((n, ncols), jnp.float32),
        ],
        scratch_types=[
            pltpu.VMEM((nch, CHUNK), jnp.int32),
            pltpu.VMEM((nch, CHUNK), jnp.int32),
            pltpu.VMEM((nch, CHUNK), jnp.float32),
            pltpu.VMEM((CHUNK, ncols), jnp.float32),
            pltpu.VMEM_SHARED((n, ncols), jnp.float32),
            pltpu.SemaphoreType.DMA,
        ],
    )
    def spmm1(h0, h1, src2d, dst2d, w2d, s0, s1,
              srcbuf, dstbuf, wbuf, rows, acc, gsem):
        c = lax.axis_index("c")
        s = lax.axis_index("s")
        row0 = s * ROWS_PER_TILE

        pltpu.sync_copy(src2d.at[pl.ds(s * nch, nch)], srcbuf)
        pltpu.sync_copy(dst2d.at[pl.ds(s * nch, nch)], dstbuf)
        pltpu.sync_copy(w2d.at[pl.ds(s * nch, nch)], wbuf)

        _zero_rows(rows, DRAIN_CHUNK, ncols)
        for j in range(ROWS_PER_TILE // DRAIN_CHUNK):
            pltpu.sync_copy(rows, acc.at[pl.ds(row0 + j * DRAIN_CHUNK, DRAIN_CHUNK)])
        plsc.subcore_barrier()

        for cidx, h_hbm in enumerate([h0, h1]):
            @pl.when(c == cidx)
            def _():
                _spmm_accumulate(h_hbm, acc, srcbuf, dstbuf, wbuf, rows,
                                 gsem, nch, ncols)

        plsc.subcore_barrier()

        for cidx, out_hbm in enumerate([s0, s1]):
            @pl.when(c == cidx)
            def _():
                _drain(acc, rows, out_hbm, row0)

    return spmm1


def _make_spmm2(n, nch, ncols):
    mesh = plsc.VectorSubcoreMesh(core_axis_name="c", subcore_axis_name="s")

    @functools.partial(
        pl.kernel,
        mesh=mesh,
        out_type=[
            jax.ShapeDtypeStruct((n, ncols), jnp.float32),
            jax.ShapeDtypeStruct((n, ncols), jnp.float32),
        ],
        scratch_types=[
            pltpu.VMEM((nch, CHUNK), jnp.int32),
            pltpu.VMEM((nch, CHUNK), jnp.int32),
            pltpu.VMEM((nch, CHUNK), jnp.float32),
            pltpu.VMEM((CHUNK, ncols), jnp.float32),
            pltpu.VMEM_SHARED((n, ncols), jnp.float32),
            pltpu.SemaphoreType.DMA,
        ],
    )
    def spmm2(h2, src2d, dst2d, w2d, p0, p1,
              srcbuf, dstbuf, wbuf, rows, acc, gsem):
        c = lax.axis_index("c")
        s = lax.axis_index("s")
        wid = c * N_SUBCORES + s
        row0 = s * ROWS_PER_TILE

        pltpu.sync_copy(src2d.at[pl.ds(wid * nch, nch)], srcbuf)
        pltpu.sync_copy(dst2d.at[pl.ds(wid * nch, nch)], dstbuf)
        pltpu.sync_copy(w2d.at[pl.ds(wid * nch, nch)], wbuf)

        _zero_rows(rows, DRAIN_CHUNK, ncols)
        for j in range(ROWS_PER_TILE // DRAIN_CHUNK):
            pltpu.sync_copy(rows, acc.at[pl.ds(row0 + j * DRAIN_CHUNK, DRAIN_CHUNK)])
        plsc.subcore_barrier()

        _spmm_accumulate(h2, acc, srcbuf, dstbuf, wbuf, rows, gsem, nch, ncols)

        plsc.subcore_barrier()

        for cidx, out_hbm in enumerate([p0, p1]):
            @pl.when(c == cidx)
            def _():
                _drain(acc, rows, out_hbm, row0)

    return spmm2


def _mm1_body(x_ref, w_ref, o0_ref, o1_ref):
    xb = x_ref[...]
    o0_ref[...] = jnp.dot(xb, w_ref[:, :128], preferred_element_type=jnp.float32)
    o1_ref[...] = jnp.dot(xb, w_ref[:, 128:], preferred_element_type=jnp.float32)


def _mm2_body(s0_ref, s1_ref, b1_ref, w2_ref, o_ref):
    a0 = jnp.maximum(s0_ref[...] + b1_ref[0, :128], 0.0)
    a1 = jnp.maximum(s1_ref[...] + b1_ref[0, 128:], 0.0)
    acc = jnp.dot(a0, w2_ref[:128, :], preferred_element_type=jnp.float32)
    acc += jnp.dot(a1, w2_ref[128:, :], preferred_element_type=jnp.float32)
    o_ref[...] = acc


def _combine_body(p0_ref, p1_ref, b2_ref, o_ref):
    o_ref[...] = p0_ref[...] + p1_ref[...] + b2_ref[0, :]


def _pad_edges(src, dst, w, n_parts, e_total):
    per = -(-e_total // (n_parts * CHUNK * 8)) * CHUNK * 8
    e_pad = per * n_parts
    pad = e_pad - e_total
    src_p = jnp.pad(src, (0, pad)).reshape(n_parts * (per // CHUNK), CHUNK)
    dst_p = jnp.pad(dst, (0, pad)).reshape(n_parts * (per // CHUNK), CHUNK)
    w_p = jnp.pad(w, (0, pad)).reshape(n_parts * (per // CHUNK), CHUNK)
    return src_p, dst_p, w_p, per // CHUNK


def kernel(x, edge_index, edge_weight, W1, b1, W2, b2):
    n, d_in = x.shape
    npad = N_PAD
    x = jnp.pad(x, ((0, npad - n), (0, 0)))
    e = edge_weight.shape[0]
    d_h = W1.shape[1]
    d_out = W2.shape[1]
    half = d_h // 2

    dst = edge_index[0].astype(jnp.int32)
    src = edge_index[1].astype(jnp.int32)
    w = edge_weight

    src1, dst1, w1, nch1 = _pad_edges(src, dst, w, N_SUBCORES, e)
    src2, dst2, w2, nch2 = _pad_edges(src, dst, w, N_SUBCORES * N_CORES, e)

    grid = (npad // MM_BLOCK,)
    h0, h1 = pl.pallas_call(
        _mm1_body,
        grid=grid,
        in_specs=[
            pl.BlockSpec((MM_BLOCK, d_in), lambda i: (i, 0)),
            pl.BlockSpec((d_in, d_h), lambda i: (0, 0)),
        ],
        out_specs=[
            pl.BlockSpec((MM_BLOCK, half), lambda i: (i, 0)),
            pl.BlockSpec((MM_BLOCK, half), lambda i: (i, 0)),
        ],
        out_shape=[
            jax.ShapeDtypeStruct((npad, half), jnp.float32),
            jax.ShapeDtypeStruct((npad, half), jnp.float32),
        ],
    )(x, W1)

    s0, s1 = _make_spmm1(npad, nch1, half)(h0, h1, src1, dst1, w1)

    h2 = pl.pallas_call(
        _mm2_body,
        grid=grid,
        in_specs=[
            pl.BlockSpec((MM_BLOCK, half), lambda i: (i, 0)),
            pl.BlockSpec((MM_BLOCK, half), lambda i: (i, 0)),
            pl.BlockSpec((1, d_h), lambda i: (0, 0)),
            pl.BlockSpec((d_h, d_out), lambda i: (0, 0)),
        ],
        out_specs=pl.BlockSpec((MM_BLOCK, d_out), lambda i: (i, 0)),
        out_shape=jax.ShapeDtypeStruct((npad, d_out), jnp.float32),
    )(s0, s1, b1.reshape(1, d_h), W2)

    p0, p1 = _make_spmm2(npad, nch2, d_out)(h2, src2, dst2, w2)

    out = pl.pallas_call(
        _combine_body,
        grid=grid,
        in_specs=[
            pl.BlockSpec((MM_BLOCK, d_out), lambda i: (i, 0)),
            pl.BlockSpec((MM_BLOCK, d_out), lambda i: (i, 0)),
            pl.BlockSpec((1, d_out), lambda i: (0, 0)),
        ],
        out_specs=pl.BlockSpec((MM_BLOCK, d_out), lambda i: (i, 0)),
        out_shape=jax.ShapeDtypeStruct((npad, d_out), jnp.float32),
    )(p0, p1, b2.reshape(1, d_out))

    return out[:n]


# speedup vs baseline: 3.3429x; 1.2000x over previous
import functools

import jax
import jax.numpy as jnp
from jax import lax
from jax.experimental import pallas as pl
from jax.experimental.pallas import tpu as pltpu
from jax.experimental.pallas import tpu_sc as plsc

N_PAD = 10240
CHUNK = 128
LANES = 16
N_SUBCORES = 16
N_CORES = 2
ROWS_PER_TILE = N_PAD // N_SUBCORES
DRAIN_CHUNK = 128
MM_BLOCK = 1024


def _zero_rows(rows_ref, nrows, ncols):
    zv = jnp.zeros((LANES,), jnp.float32)

    def body(i, _):
        for k in range(ncols // LANES):
            rows_ref[i, pl.ds(k * LANES, LANES)] = zv
        return 0

    lax.fori_loop(0, nrows, body, 0)


def _spmm_accumulate(h_hbm, acc, src2d, dst2d, w2d, base, srcbuf, dstbuf,
                     wbuf, rows0, rows1, sem0, sem1, esem, nch, ncols, bst):
    nblk = nch // bst

    def stage(b, pb, copy):
        sl = pl.ds(base + b * bst, bst)
        copy(src2d.at[sl], srcbuf.at[pb])
        copy(dst2d.at[sl], dstbuf.at[pb])
        copy(w2d.at[sl], wbuf.at[pb])

    def start(pb, t, buf, sem):
        pltpu.async_copy(h_hbm.at[srcbuf.at[pb, t]], buf, sem)

    def finish(pb, t, buf, sem):
        pltpu.make_async_copy(h_hbm.at[srcbuf.at[pb, t]], buf, sem).wait()

    def scale(pb, t, buf):
        def edge_group(g, _):
            bs = g * LANES
            wvec = wbuf[pb, t, pl.ds(g * LANES, LANES)]
            for i in range(LANES):
                wv = wvec[i]
                for k in range(ncols // LANES):
                    sl = pl.ds(k * LANES, LANES)
                    buf[bs + i, sl] = buf[bs + i, sl] * wv
            return 0

        lax.fori_loop(0, CHUNK // LANES, edge_group, 0)

    def step(pb, t, buf, sem):
        finish(pb, t, buf, sem)
        scale(pb, t, buf)
        pltpu.sync_copy(buf, acc.at[dstbuf.at[pb, t]], add=True)

        @pl.when(t + 2 < bst)
        def _():
            start(pb, t + 2, buf, sem)

    stage(0, 0, pltpu.sync_copy)
    start(0, 0, rows0, sem0)
    start(0, 1, rows1, sem1)

    for b in range(nblk):
        pb = b & 1
        if b + 1 < nblk:
            stage(b + 1, pb ^ 1, lambda s, d: pltpu.async_copy(s, d, esem))

        def inner(tt, _):
            step(pb, 2 * tt, rows0, sem0)
            step(pb, 2 * tt + 1, rows1, sem1)
            return 0

        lax.fori_loop(0, bst // 2, inner, 0)

        if b + 1 < nblk:
            stage(b + 1, pb ^ 1,
                  lambda s, d: pltpu.make_async_copy(s, d, esem).wait())
            start(pb ^ 1, 0, rows0, sem0)
            start(pb ^ 1, 1, rows1, sem1)


def _drain(acc, dbuf, out_hbm, row0):
    for j in range(ROWS_PER_TILE // DRAIN_CHUNK):
        r = row0 + j * DRAIN_CHUNK
        pltpu.sync_copy(acc.at[pl.ds(r, DRAIN_CHUNK)], dbuf)
        pltpu.sync_copy(dbuf, out_hbm.at[pl.ds(r, DRAIN_CHUNK)])


def _spmm_scratch(n, nch, ncols, bst):
    return [
        pltpu.VMEM((2, bst, CHUNK), jnp.int32),
        pltpu.VMEM((2, bst, CHUNK), jnp.int32),
        pltpu.VMEM((2, bst, CHUNK), jnp.float32),
        pltpu.VMEM((CHUNK, ncols), jnp.float32),
        pltpu.VMEM((CHUNK, ncols), jnp.float32),
        pltpu.VMEM_SHARED((n, ncols), jnp.float32),
        pltpu.SemaphoreType.DMA,
        pltpu.SemaphoreType.DMA,
        pltpu.SemaphoreType.DMA,
    ]


def _make_spmm1(n, nch, ncols, bst):
    mesh = plsc.VectorSubcoreMesh(core_axis_name="c", subcore_axis_name="s")

    @functools.partial(
        pl.kernel,
        mesh=mesh,
        out_type=[
            jax.ShapeDtypeStruct((n, ncols), jnp.float32),
            jax.ShapeDtypeStruct((n, ncols), jnp.float32),
        ],
        scratch_types=_spmm_scratch(n, nch, ncols, bst),
    )
    def spmm1(h0, h1, src2d, dst2d, w2d, s0, s1,
              srcbuf, dstbuf, wbuf, rows0, rows1, acc, sem0, sem1, esem):
        c = lax.axis_index("c")
        s = lax.axis_index("s")
        row0 = s * ROWS_PER_TILE

        _zero_rows(rows0, DRAIN_CHUNK, ncols)
        for j in range(ROWS_PER_TILE // DRAIN_CHUNK):
            pltpu.sync_copy(rows0, acc.at[pl.ds(row0 + j * DRAIN_CHUNK, DRAIN_CHUNK)])
        plsc.subcore_barrier()

        for cidx, h_hbm in enumerate([h0, h1]):
            @pl.when(c == cidx)
            def _():
                _spmm_accumulate(h_hbm, acc, src2d, dst2d, w2d, s * nch,
                                 srcbuf, dstbuf, wbuf, rows0, rows1,
                                 sem0, sem1, esem, nch, ncols, bst)

        plsc.subcore_barrier()

        for cidx, out_hbm in enumerate([s0, s1]):
            @pl.when(c == cidx)
            def _():
                _drain(acc, rows0, out_hbm, row0)

    return spmm1


def _make_spmm2(n, nch, ncols, bst):
    mesh = plsc.VectorSubcoreMesh(core_axis_name="c", subcore_axis_name="s")

    @functools.partial(
        pl.kernel,
        mesh=mesh,
        out_type=[
            jax.ShapeDtypeStruct((n, ncols), jnp.float32),
            jax.ShapeDtypeStruct((n, ncols), jnp.float32),
        ],
        scratch_types=_spmm_scratch(n, nch, ncols, bst),
    )
    def spmm2(h2, src2d, dst2d, w2d, p0, p1,
              srcbuf, dstbuf, wbuf, rows0, rows1, acc, sem0, sem1, esem):
        c = lax.axis_index("c")
        s = lax.axis_index("s")
        wid = c * N_SUBCORES + s
        row0 = s * ROWS_PER_TILE

        _zero_rows(rows0, DRAIN_CHUNK, ncols)
        for j in range(ROWS_PER_TILE // DRAIN_CHUNK):
            pltpu.sync_copy(rows0, acc.at[pl.ds(row0 + j * DRAIN_CHUNK, DRAIN_CHUNK)])
        plsc.subcore_barrier()

        _spmm_accumulate(h2, acc, src2d, dst2d, w2d, wid * nch,
                         srcbuf, dstbuf, wbuf, rows0, rows1,
                         sem0, sem1, esem, nch, ncols, bst)

        plsc.subcore_barrier()

        for cidx, out_hbm in enumerate([p0, p1]):
            @pl.when(c == cidx)
            def _():
                _drain(acc, rows0, out_hbm, row0)

    return spmm2


def _mm1_body(x_ref, w_ref, o0_ref, o1_ref):
    xb = x_ref[...]
    o0_ref[...] = jnp.dot(xb, w_ref[:, :128], preferred_element_type=jnp.float32)
    o1_ref[...] = jnp.dot(xb, w_ref[:, 128:], preferred_element_type=jnp.float32)


def _mm2_body(s0_ref, s1_ref, b1_ref, w2_ref, o_ref):
    a0 = jnp.maximum(s0_ref[...] + b1_ref[0, :128], 0.0)
    a1 = jnp.maximum(s1_ref[...] + b1_ref[0, 128:], 0.0)
    acc = jnp.dot(a0, w2_ref[:128, :], preferred_element_type=jnp.float32)
    acc += jnp.dot(a1, w2_ref[128:, :], preferred_element_type=jnp.float32)
    o_ref[...] = acc


def _combine_body(p0_ref, p1_ref, b2_ref, o_ref):
    o_ref[...] = p0_ref[...] + p1_ref[...] + b2_ref[0, :]


def _pad_edges(src, dst, w, n_parts, e_total, bst):
    blk = CHUNK * bst
    per = -(-e_total // (n_parts * blk)) * blk
    e_pad = per * n_parts
    pad = e_pad - e_total
    src_p = jnp.pad(src, (0, pad)).reshape(e_pad // CHUNK, CHUNK)
    dst_p = jnp.pad(dst, (0, pad)).reshape(e_pad // CHUNK, CHUNK)
    w_p = jnp.pad(w, (0, pad)).reshape(e_pad // CHUNK, CHUNK)
    return src_p, dst_p, w_p, per // CHUNK


def kernel(x, edge_index, edge_weight, W1, b1, W2, b2):
    n, d_in = x.shape
    npad = N_PAD
    x = jnp.pad(x, ((0, npad - n), (0, 0)))
    e = edge_weight.shape[0]
    d_h = W1.shape[1]
    d_out = W2.shape[1]
    half = d_h // 2
    bst1, bst2 = 16, 8

    dst = edge_index[0].astype(jnp.int32)
    src = edge_index[1].astype(jnp.int32)
    w = edge_weight

    src1, dst1, w1, nch1 = _pad_edges(src, dst, w, N_SUBCORES, e, bst1)
    src2, dst2, w2, nch2 = _pad_edges(src, dst, w, N_SUBCORES * N_CORES, e, bst2)

    grid = (npad // MM_BLOCK,)
    h0, h1 = pl.pallas_call(
        _mm1_body,
        grid=grid,
        in_specs=[
            pl.BlockSpec((MM_BLOCK, d_in), lambda i: (i, 0)),
            pl.BlockSpec((d_in, d_h), lambda i: (0, 0)),
        ],
        out_specs=[
            pl.BlockSpec((MM_BLOCK, half), lambda i: (i, 0)),
            pl.BlockSpec((MM_BLOCK, half), lambda i: (i, 0)),
        ],
        out_shape=[
            jax.ShapeDtypeStruct((npad, half), jnp.float32),
            jax.ShapeDtypeStruct((npad, half), jnp.float32),
        ],
    )(x, W1)

    s0, s1 = _make_spmm1(npad, nch1, half, bst1)(h0, h1, src1, dst1, w1)

    h2 = pl.pallas_call(
        _mm2_body,
        grid=grid,
        in_specs=[
            pl.BlockSpec((MM_BLOCK, half), lambda i: (i, 0)),
            pl.BlockSpec((MM_BLOCK, half), lambda i: (i, 0)),
            pl.BlockSpec((1, d_h), lambda i: (0, 0)),
            pl.BlockSpec((d_h, d_out), lambda i: (0, 0)),
        ],
        out_specs=pl.BlockSpec((MM_BLOCK, d_out), lambda i: (i, 0)),
        out_shape=jax.ShapeDtypeStruct((npad, d_out), jnp.float32),
    )(s0, s1, b1.reshape(1, d_h), W2)

    p0, p1 = _make_spmm2(npad, nch2, d_out, bst2)(h2, src2, dst2, w2)

    out = pl.pallas_call(
        _combine_body,
        grid=grid,
        in_specs=[
            pl.BlockSpec((MM_BLOCK, d_out), lambda i: (i, 0)),
            pl.BlockSpec((MM_BLOCK, d_out), lambda i: (i, 0)),
            pl.BlockSpec((1, d_out), lambda i: (0, 0)),
        ],
        out_specs=pl.BlockSpec((MM_BLOCK, d_out), lambda i: (i, 0)),
        out_shape=jax.ShapeDtypeStruct((npad, d_out), jnp.float32),
    )(p0, p1, b2.reshape(1, d_out))

    return out[:n]


# speedup vs baseline: 7.2037x; 2.1549x over previous
import functools

import jax
import jax.numpy as jnp
from jax import lax
from jax.experimental import pallas as pl
from jax.experimental.pallas import tpu as pltpu
from jax.experimental.pallas import tpu_sc as plsc

N_PAD = 10240
CHUNK = 128
LANES = 16
N_SUBCORES = 16
N_CORES = 2
ROWS_PER_TILE = N_PAD // N_SUBCORES
DRAIN_CHUNK = 128
MM_BLOCK = 1024


def _zero_rows(rows_ref, nrows, ncols):
    zv = jnp.zeros((LANES,), jnp.float32)

    def body(i, _):
        for k in range(ncols // LANES):
            rows_ref[i, pl.ds(k * LANES, LANES)] = zv
        return 0

    lax.fori_loop(0, nrows, body, 0)


def _spmm_accumulate(h_hbm, acc, src2d, dst2d, w2d, base, srcbuf, dstbuf,
                     wbuf, rows0, rows1, sem0, sem1, esem, nch, ncols, bst):
    nblk = nch // bst

    def stage(b, pb, copy):
        sl = pl.ds(base + b * bst, bst)
        copy(src2d.at[sl], srcbuf.at[pb])
        copy(dst2d.at[sl], dstbuf.at[pb])
        copy(w2d.at[sl], wbuf.at[pb])

    def start(pb, t, buf, sem):
        pltpu.async_copy(h_hbm.at[srcbuf.at[pb, t]], buf, sem)

    def finish(pb, t, buf, sem):
        pltpu.make_async_copy(h_hbm.at[srcbuf.at[pb, t]], buf, sem).wait()

    def scale(pb, t, buf):
        def edge_group(g, _):
            bs = g * LANES
            wvec = wbuf[pb, t, pl.ds(g * LANES, LANES)]
            for i in range(LANES):
                wv = wvec[i]
                for k in range(ncols // LANES):
                    sl = pl.ds(k * LANES, LANES)
                    buf[bs + i, sl] = buf[bs + i, sl] * wv
            return 0

        lax.fori_loop(0, CHUNK // LANES, edge_group, 0)

    def step(pb, t, buf, sem):
        finish(pb, t, buf, sem)
        scale(pb, t, buf)
        pltpu.sync_copy(buf, acc.at[dstbuf.at[pb, t]], add=True)

        @pl.when(t + 2 < bst)
        def _():
            start(pb, t + 2, buf, sem)

    stage(0, 0, pltpu.sync_copy)
    start(0, 0, rows0, sem0)
    start(0, 1, rows1, sem1)

    for b in range(nblk):
        pb = b & 1
        if b + 1 < nblk:
            stage(b + 1, pb ^ 1, lambda s, d: pltpu.async_copy(s, d, esem))

        def inner(tt, _):
            step(pb, 2 * tt, rows0, sem0)
            step(pb, 2 * tt + 1, rows1, sem1)
            return 0

        lax.fori_loop(0, bst // 2, inner, 0)

        if b + 1 < nblk:
            stage(b + 1, pb ^ 1,
                  lambda s, d: pltpu.make_async_copy(s, d, esem).wait())
            start(pb ^ 1, 0, rows0, sem0)
            start(pb ^ 1, 1, rows1, sem1)


def _drain(acc, dbuf, out_hbm, row0):
    for j in range(ROWS_PER_TILE // DRAIN_CHUNK):
        r = row0 + j * DRAIN_CHUNK
        pltpu.sync_copy(acc.at[pl.ds(r, DRAIN_CHUNK)], dbuf)
        pltpu.sync_copy(dbuf, out_hbm.at[pl.ds(r, DRAIN_CHUNK)])


def _spmm_scratch(n, nch, ncols, bst):
    return [
        pltpu.VMEM((2, bst, CHUNK), jnp.int32),
        pltpu.VMEM((2, bst, CHUNK), jnp.int32),
        pltpu.VMEM((2, bst, CHUNK), jnp.float32),
        pltpu.VMEM((CHUNK, ncols), jnp.float32),
        pltpu.VMEM((CHUNK, ncols), jnp.float32),
        pltpu.VMEM_SHARED((n, ncols), jnp.float32),
        pltpu.SemaphoreType.DMA,
        pltpu.SemaphoreType.DMA,
        pltpu.SemaphoreType.DMA,
    ]


def _make_spmm1(n, nch, ncols, bst):
    mesh = plsc.VectorSubcoreMesh(core_axis_name="c", subcore_axis_name="s")

    @functools.partial(
        pl.kernel,
        mesh=mesh,
        out_type=[
            jax.ShapeDtypeStruct((n, ncols), jnp.float32),
            jax.ShapeDtypeStruct((n, ncols), jnp.float32),
        ],
        scratch_types=_spmm_scratch(n, nch, ncols, bst),
    )
    def spmm1(h0, h1, src2d, dst2d, w2d, s0, s1,
              srcbuf, dstbuf, wbuf, rows0, rows1, acc, sem0, sem1, esem):
        c = lax.axis_index("c")
        s = lax.axis_index("s")
        row0 = s * ROWS_PER_TILE

        _zero_rows(rows0, DRAIN_CHUNK, ncols)
        for j in range(ROWS_PER_TILE // DRAIN_CHUNK):
            pltpu.sync_copy(rows0, acc.at[pl.ds(row0 + j * DRAIN_CHUNK, DRAIN_CHUNK)])
        plsc.subcore_barrier()

        for cidx, h_hbm in enumerate([h0, h1]):
            @pl.when(c == cidx)
            def _():
                _spmm_accumulate(h_hbm, acc, src2d, dst2d, w2d, s * nch,
                                 srcbuf, dstbuf, wbuf, rows0, rows1,
                                 sem0, sem1, esem, nch, ncols, bst)

        plsc.subcore_barrier()

        for cidx, out_hbm in enumerate([s0, s1]):
            @pl.when(c == cidx)
            def _():
                _drain(acc, rows0, out_hbm, row0)

    return spmm1


def _make_spmm2(n, nch, ncols, bst):
    mesh = plsc.VectorSubcoreMesh(core_axis_name="c", subcore_axis_name="s")

    @functools.partial(
        pl.kernel,
        mesh=mesh,
        out_type=[
            jax.ShapeDtypeStruct((n, ncols), jnp.float32),
            jax.ShapeDtypeStruct((n, ncols), jnp.float32),
        ],
        scratch_types=_spmm_scratch(n, nch, ncols, bst),
    )
    def spmm2(h2, src2d, dst2d, w2d, p0, p1,
              srcbuf, dstbuf, wbuf, rows0, rows1, acc, sem0, sem1, esem):
        c = lax.axis_index("c")
        s = lax.axis_index("s")
        wid = c * N_SUBCORES + s
        row0 = s * ROWS_PER_TILE

        _zero_rows(rows0, DRAIN_CHUNK, ncols)
        for j in range(ROWS_PER_TILE // DRAIN_CHUNK):
            pltpu.sync_copy(rows0, acc.at[pl.ds(row0 + j * DRAIN_CHUNK, DRAIN_CHUNK)])
        plsc.subcore_barrier()

        _spmm_accumulate(h2, acc, src2d, dst2d, w2d, wid * nch,
                         srcbuf, dstbuf, wbuf, rows0, rows1,
                         sem0, sem1, esem, nch, ncols, bst)

        plsc.subcore_barrier()

        for cidx, out_hbm in enumerate([p0, p1]):
            @pl.when(c == cidx)
            def _():
                _drain(acc, rows0, out_hbm, row0)

    return spmm2


def _mm1_body(x_ref, w_ref, o0_ref, o1_ref):
    xb = x_ref[...]
    o0_ref[...] = jnp.dot(xb, w_ref[:, :128], preferred_element_type=jnp.float32)
    o1_ref[...] = jnp.dot(xb, w_ref[:, 128:], preferred_element_type=jnp.float32)


def _mm2_body(s0_ref, s1_ref, b1_ref, w2_ref, o_ref):
    a0 = jnp.maximum(s0_ref[...] + b1_ref[0, :128], 0.0)
    a1 = jnp.maximum(s1_ref[...] + b1_ref[0, 128:], 0.0)
    acc = jnp.dot(a0, w2_ref[:128, :], preferred_element_type=jnp.float32)
    acc += jnp.dot(a1, w2_ref[128:, :], preferred_element_type=jnp.float32)
    o_ref[...] = acc


def _combine_body(p0_ref, p1_ref, b2_ref, o_ref):
    o_ref[...] = p0_ref[...] + p1_ref[...] + b2_ref[0, :]


def _pad_edges(src, dst, w, n_parts, e_total, bst, n, npad):
    blk = CHUNK * bst
    per = -(-e_total // (n_parts * blk)) * blk
    e_pad = per * n_parts
    pad = e_pad - e_total
    pidx = jnp.arange(pad, dtype=jnp.int32)
    src_p = jnp.concatenate([src, pidx % n]).reshape(e_pad // CHUNK, CHUNK)
    dst_p = jnp.concatenate([dst, n + pidx % (npad - n)]).reshape(e_pad // CHUNK, CHUNK)
    w_p = jnp.concatenate([w, jnp.zeros((pad,), w.dtype)]).reshape(e_pad // CHUNK, CHUNK)
    return src_p, dst_p, w_p, per // CHUNK


def kernel(x, edge_index, edge_weight, W1, b1, W2, b2):
    n, d_in = x.shape
    npad = N_PAD
    x = jnp.pad(x, ((0, npad - n), (0, 0)))
    e = edge_weight.shape[0]
    d_h = W1.shape[1]
    d_out = W2.shape[1]
    half = d_h // 2
    bst1, bst2 = 16, 8

    dst = edge_index[0].astype(jnp.int32)
    src = edge_index[1].astype(jnp.int32)
    w = edge_weight

    src1, dst1, w1, nch1 = _pad_edges(src, dst, w, N_SUBCORES, e, bst1, n, npad)
    src2, dst2, w2, nch2 = _pad_edges(src, dst, w, N_SUBCORES * N_CORES, e, bst2, n, npad)

    grid = (npad // MM_BLOCK,)
    h0, h1 = pl.pallas_call(
        _mm1_body,
        grid=grid,
        in_specs=[
            pl.BlockSpec((MM_BLOCK, d_in), lambda i: (i, 0)),
            pl.BlockSpec((d_in, d_h), lambda i: (0, 0)),
        ],
        out_specs=[
            pl.BlockSpec((MM_BLOCK, half), lambda i: (i, 0)),
            pl.BlockSpec((MM_BLOCK, half), lambda i: (i, 0)),
        ],
        out_shape=[
            jax.ShapeDtypeStruct((npad, half), jnp.float32),
            jax.ShapeDtypeStruct((npad, half), jnp.float32),
        ],
    )(x, W1)

    s0, s1 = _make_spmm1(npad, nch1, half, bst1)(h0, h1, src1, dst1, w1)

    h2 = pl.pallas_call(
        _mm2_body,
        grid=grid,
        in_specs=[
            pl.BlockSpec((MM_BLOCK, half), lambda i: (i, 0)),
            pl.BlockSpec((MM_BLOCK, half), lambda i: (i, 0)),
            pl.BlockSpec((1, d_h), lambda i: (0, 0)),
            pl.BlockSpec((d_h, d_out), lambda i: (0, 0)),
        ],
        out_specs=pl.BlockSpec((MM_BLOCK, d_out), lambda i: (i, 0)),
        out_shape=jax.ShapeDtypeStruct((npad, d_out), jnp.float32),
    )(s0, s1, b1.reshape(1, d_h), W2)

    p0, p1 = _make_spmm2(npad, nch2, d_out, bst2)(h2, src2, dst2, w2)

    out = pl.pallas_call(
        _combine_body,
        grid=grid,
        in_specs=[
            pl.BlockSpec((MM_BLOCK, d_out), lambda i: (i, 0)),
            pl.BlockSpec((MM_BLOCK, d_out), lambda i: (i, 0)),
            pl.BlockSpec((1, d_out), lambda i: (0, 0)),
        ],
        out_specs=pl.BlockSpec((MM_BLOCK, d_out), lambda i: (i, 0)),
        out_shape=jax.ShapeDtypeStruct((npad, d_out), jnp.float32),
    )(p0, p1, b2.reshape(1, d_out))

    return out[:n]


# speedup vs baseline: 8.5650x; 1.1890x over previous
import functools

import jax
import jax.numpy as jnp
from jax import lax
from jax.experimental import pallas as pl
from jax.experimental.pallas import tpu as pltpu
from jax.experimental.pallas import tpu_sc as plsc

N_PAD = 10240
CHUNK = 80
LANES = 16
N_SUBCORES = 16
N_CORES = 2
ROWS_PER_TILE = N_PAD // N_SUBCORES
DRAIN_CHUNK = 80
MM_BLOCK = 1000


def _zero_rows(rows_ref, nrows, ncols):
    zv = jnp.zeros((LANES,), jnp.float32)

    def body(i, _):
        for k in range(ncols // LANES):
            rows_ref[i, pl.ds(k * LANES, LANES)] = zv
        return 0

    lax.fori_loop(0, nrows, body, 0)


def _spmm_accumulate(h_hbm, acc, src2d, dst2d, w2d, base, srcbuf, dstbuf,
                     wbuf, bufs, gsems, ssems, esem, nch, ncols, bst):
    nblk = nch // bst
    nbuf = len(bufs)

    def stage(b, pb, copy):
        sl = pl.ds(base + b * bst, bst)
        copy(src2d.at[sl], srcbuf.at[pb])
        copy(dst2d.at[sl], dstbuf.at[pb])
        copy(w2d.at[sl], wbuf.at[pb])

    def start(pb, t, buf, sem):
        pltpu.async_copy(h_hbm.at[srcbuf.at[pb, t]], buf, sem)

    def finish(pb, t, buf, sem):
        pltpu.make_async_copy(h_hbm.at[srcbuf.at[pb, t]], buf, sem).wait()

    def scat_start(pb, t, buf, sem):
        pltpu.async_copy(buf, acc.at[dstbuf.at[pb, t]], sem, add=True)

    def scat_wait(pb, t, buf, sem):
        pltpu.make_async_copy(buf, acc.at[dstbuf.at[pb, t]], sem).wait()

    def scale(pb, t, buf):
        def edge_group(g, _):
            bs = g * LANES
            wvec = wbuf[pb, t, pl.ds(g * LANES, LANES)]
            for i in range(LANES):
                wv = wvec[i]
                for k in range(ncols // LANES):
                    sl = pl.ds(k * LANES, LANES)
                    buf[bs + i, sl] = buf[bs + i, sl] * wv
            return 0

        lax.fori_loop(0, CHUNK // LANES, edge_group, 0)

    stage(0, 0, pltpu.sync_copy)
    for t in range(nbuf - 1):
        start(0, t, bufs[t], gsems[t])

    def block(b, _):
        pb = b % 2
        pb_next = (b + 1) % 2
        has_next = b < nblk - 1

        @pl.when(has_next)
        def _():
            stage(b + 1, pb_next, lambda s, d: pltpu.async_copy(s, d, esem))

        for t in range(bst):
            bi = t % nbuf
            pi = (t + nbuf - 1) % nbuf
            finish(pb, t, bufs[bi], gsems[bi])
            scale(pb, t, bufs[bi])
            if t == 0:
                @pl.when(b > 0)
                def _():
                    scat_wait(pb, t, bufs[pi], ssems[pi])
            else:
                scat_wait(pb, t, bufs[pi], ssems[pi])
            scat_start(pb, t, bufs[bi], ssems[bi])
            nt = t + nbuf - 1
            if nt < bst:
                start(pb, nt, bufs[nt % nbuf], gsems[nt % nbuf])
            else:
                if nt == bst:
                    @pl.when(has_next)
                    def _():
                        stage(b + 1, pb_next,
                              lambda s, d: pltpu.make_async_copy(s, d, esem).wait())

                @pl.when(has_next)
                def _():
                    start(pb_next, nt - bst, bufs[nt % nbuf], gsems[nt % nbuf])
        return 0

    lax.fori_loop(0, nblk, block, 0)
    scat_wait(0, bst - 1, bufs[(bst - 1) % nbuf], ssems[(bst - 1) % nbuf])


def _drain(acc, dbuf, out_hbm, row0):
    for j in range(ROWS_PER_TILE // DRAIN_CHUNK):
        r = row0 + j * DRAIN_CHUNK
        pltpu.sync_copy(acc.at[pl.ds(r, DRAIN_CHUNK)], dbuf)
        pltpu.sync_copy(dbuf, out_hbm.at[pl.ds(r, DRAIN_CHUNK)])


def _spmm_scratch(n, nch, ncols, bst):
    return [
        pltpu.VMEM((2, bst, CHUNK), jnp.int32),
        pltpu.VMEM((2, bst, CHUNK), jnp.int32),
        pltpu.VMEM((2, bst, CHUNK), jnp.float32),
        pltpu.VMEM((CHUNK, ncols), jnp.float32),
        pltpu.VMEM((CHUNK, ncols), jnp.float32),
        pltpu.VMEM((CHUNK, ncols), jnp.float32),
        pltpu.VMEM((CHUNK, ncols), jnp.float32),
        pltpu.VMEM_SHARED((n, ncols), jnp.float32),
    ] + [pltpu.SemaphoreType.DMA] * 9


def _make_spmm1(n, nch, ncols, bst):
    mesh = plsc.VectorSubcoreMesh(core_axis_name="c", subcore_axis_name="s")

    @functools.partial(
        pl.kernel,
        mesh=mesh,
        out_type=[
            jax.ShapeDtypeStruct((n, ncols), jnp.float32),
            jax.ShapeDtypeStruct((n, ncols), jnp.float32),
        ],
        scratch_types=_spmm_scratch(n, nch, ncols, bst),
    )
    def spmm1(h0, h1, src2d, dst2d, w2d, s0, s1,
              srcbuf, dstbuf, wbuf, r0, r1, r2, r3, acc, *sems):
        c = lax.axis_index("c")
        s = lax.axis_index("s")
        row0 = s * ROWS_PER_TILE

        _zero_rows(r0, DRAIN_CHUNK, ncols)
        for j in range(ROWS_PER_TILE // DRAIN_CHUNK):
            pltpu.sync_copy(r0, acc.at[pl.ds(row0 + j * DRAIN_CHUNK, DRAIN_CHUNK)])
        plsc.subcore_barrier()

        for cidx, h_hbm in enumerate([h0, h1]):
            @pl.when(c == cidx)
            def _():
                _spmm_accumulate(h_hbm, acc, src2d, dst2d, w2d, s * nch,
                                 srcbuf, dstbuf, wbuf, [r0, r1, r2, r3],
                                 sems[0:4], sems[4:8], sems[8], nch, ncols, bst)

        plsc.subcore_barrier()

        for cidx, out_hbm in enumerate([s0, s1]):
            @pl.when(c == cidx)
            def _():
                _drain(acc, r0, out_hbm, row0)

    return spmm1


def _make_spmm2(n, nch, ncols, bst):
    mesh = plsc.VectorSubcoreMesh(core_axis_name="c", subcore_axis_name="s")

    @functools.partial(
        pl.kernel,
        mesh=mesh,
        out_type=[
            jax.ShapeDtypeStruct((n, ncols), jnp.float32),
            jax.ShapeDtypeStruct((n, ncols), jnp.float32),
        ],
        scratch_types=_spmm_scratch(n, nch, ncols, bst),
    )
    def spmm2(h2, src2d, dst2d, w2d, p0, p1,
              srcbuf, dstbuf, wbuf, r0, r1, r2, r3, acc, *sems):
        c = lax.axis_index("c")
        s = lax.axis_index("s")
        wid = c * N_SUBCORES + s
        row0 = s * ROWS_PER_TILE

        _zero_rows(r0, DRAIN_CHUNK, ncols)
        for j in range(ROWS_PER_TILE // DRAIN_CHUNK):
            pltpu.sync_copy(r0, acc.at[pl.ds(row0 + j * DRAIN_CHUNK, DRAIN_CHUNK)])
        plsc.subcore_barrier()

        _spmm_accumulate(h2, acc, src2d, dst2d, w2d, wid * nch,
                         srcbuf, dstbuf, wbuf, [r0, r1, r2, r3],
                         sems[0:4], sems[4:8], sems[8], nch, ncols, bst)

        plsc.subcore_barrier()

        for cidx, out_hbm in enumerate([p0, p1]):
            @pl.when(c == cidx)
            def _():
                _drain(acc, r0, out_hbm, row0)

    return spmm2


def _mm1_body(x_ref, w_ref, o0_ref, o1_ref):
    xb = x_ref[...]
    o0_ref[...] = jnp.dot(xb, w_ref[:, :128], preferred_element_type=jnp.float32)
    o1_ref[...] = jnp.dot(xb, w_ref[:, 128:], preferred_element_type=jnp.float32)


def _mm2_body(s0_ref, s1_ref, b1_ref, w2_ref, o_ref):
    a0 = jnp.maximum(s0_ref[...] + b1_ref[0, :128], 0.0)
    a1 = jnp.maximum(s1_ref[...] + b1_ref[0, 128:], 0.0)
    acc = jnp.dot(a0, w2_ref[:128, :], preferred_element_type=jnp.float32)
    acc += jnp.dot(a1, w2_ref[128:, :], preferred_element_type=jnp.float32)
    o_ref[...] = acc


def _combine_body(p0_ref, p1_ref, b2_ref, o_ref):
    o_ref[...] = p0_ref[...] + p1_ref[...] + b2_ref[0, :]


def _pad_edges(src, dst, w, n_parts, e_total, bst, n, npad):
    blk = CHUNK * bst
    per = -(-e_total // (n_parts * blk)) * blk
    e_pad = per * n_parts
    pad = e_pad - e_total
    pidx = jnp.arange(pad, dtype=jnp.int32)
    src_p = jnp.concatenate([src, pidx % n]).reshape(e_pad // CHUNK, CHUNK)
    dst_p = jnp.concatenate([dst, n + pidx % (npad - n)]).reshape(e_pad // CHUNK, CHUNK)
    w_p = jnp.concatenate([w, jnp.zeros((pad,), w.dtype)]).reshape(e_pad // CHUNK, CHUNK)
    return src_p, dst_p, w_p, per // CHUNK


def kernel(x, edge_index, edge_weight, W1, b1, W2, b2):
    n, d_in = x.shape
    npad = N_PAD
    e = edge_weight.shape[0]
    d_h = W1.shape[1]
    d_out = W2.shape[1]
    half = d_h // 2
    bst1, bst2 = 8, 8

    dst = edge_index[0].astype(jnp.int32)
    src = edge_index[1].astype(jnp.int32)
    w = edge_weight

    src1, dst1, w1, nch1 = _pad_edges(src, dst, w, N_SUBCORES, e, bst1, n, npad)
    src2, dst2, w2, nch2 = _pad_edges(src, dst, w, N_SUBCORES * N_CORES, e, bst2, n, npad)

    grid = (n // MM_BLOCK,)
    h0, h1 = pl.pallas_call(
        _mm1_body,
        grid=grid,
        in_specs=[
            pl.BlockSpec((MM_BLOCK, d_in), lambda i: (i, 0)),
            pl.BlockSpec((d_in, d_h), lambda i: (0, 0)),
        ],
        out_specs=[
            pl.BlockSpec((MM_BLOCK, half), lambda i: (i, 0)),
            pl.BlockSpec((MM_BLOCK, half), lambda i: (i, 0)),
        ],
        out_shape=[
            jax.ShapeDtypeStruct((n, half), jnp.float32),
            jax.ShapeDtypeStruct((n, half), jnp.float32),
        ],
    )(x, W1)

    s0, s1 = _make_spmm1(npad, nch1, half, bst1)(h0, h1, src1, dst1, w1)

    h2 = pl.pallas_call(
        _mm2_body,
        grid=grid,
        in_specs=[
            pl.BlockSpec((MM_BLOCK, half), lambda i: (i, 0)),
            pl.BlockSpec((MM_BLOCK, half), lambda i: (i, 0)),
            pl.BlockSpec((1, d_h), lambda i: (0, 0)),
            pl.BlockSpec((d_h, d_out), lambda i: (0, 0)),
        ],
        out_specs=pl.BlockSpec((MM_BLOCK, d_out), lambda i: (i, 0)),
        out_shape=jax.ShapeDtypeStruct((n, d_out), jnp.float32),
    )(s0, s1, b1.reshape(1, d_h), W2)

    p0, p1 = _make_spmm2(npad, nch2, d_out, bst2)(h2, src2, dst2, w2)

    out = pl.pallas_call(
        _combine_body,
        grid=grid,
        in_specs=[
            pl.BlockSpec((MM_BLOCK, d_out), lambda i: (i, 0)),
            pl.BlockSpec((MM_BLOCK, d_out), lambda i: (i, 0)),
            pl.BlockSpec((1, d_out), lambda i: (0, 0)),
        ],
        out_specs=pl.BlockSpec((MM_BLOCK, d_out), lambda i: (i, 0)),
        out_shape=jax.ShapeDtypeStruct((n, d_out), jnp.float32),
    )(p0, p1, b2.reshape(1, d_out))

    return out


# speedup vs baseline: 8.5720x; 1.0008x over previous
import functools

import jax
import jax.numpy as jnp
from jax import lax
from jax.experimental import pallas as pl
from jax.experimental.pallas import tpu as pltpu
from jax.experimental.pallas import tpu_sc as plsc

N_PAD = 10240
CHUNK = 80
LANES = 16
N_SUBCORES = 16
N_CORES = 2
ROWS_PER_TILE = N_PAD // N_SUBCORES
DRAIN_CHUNK = 80
MM_BLOCK = 1000


def _zero_rows(rows_ref, nrows, ncols):
    zv = jnp.zeros((LANES,), jnp.float32)

    def body(i, _):
        for k in range(ncols // LANES):
            rows_ref[i, pl.ds(k * LANES, LANES)] = zv
        return 0

    lax.fori_loop(0, nrows, body, 0)


def _spmm_accumulate(h_hbm, acc, src2d, dst2d, w2d, base, srcbuf, dstbuf,
                     wbuf, bufs, gsems, ssems, esem, nch, ncols, bst):
    nblk = nch // bst
    nbuf = len(bufs)

    def stage(b, pb, copy):
        sl = pl.ds(base + b * bst, bst)
        copy(src2d.at[sl], srcbuf.at[pb])
        copy(dst2d.at[sl], dstbuf.at[pb])
        copy(w2d.at[sl], wbuf.at[pb])

    def start(pb, t, buf, sem):
        pltpu.async_copy(h_hbm.at[srcbuf.at[pb, t]], buf, sem)

    def finish(pb, t, buf, sem):
        pltpu.make_async_copy(h_hbm.at[srcbuf.at[pb, t]], buf, sem).wait()

    def scat_start(pb, t, buf, sem):
        pltpu.async_copy(buf, acc.at[dstbuf.at[pb, t]], sem, add=True)

    def scat_wait(pb, t, buf, sem):
        pltpu.make_async_copy(buf, acc.at[dstbuf.at[pb, t]], sem).wait()

    def scale(pb, t, buf):
        def edge_group(g, _):
            bs = g * LANES
            wvec = wbuf[pb, t, pl.ds(g * LANES, LANES)]
            for i in range(LANES):
                wv = wvec[i]
                for k in range(ncols // LANES):
                    sl = pl.ds(k * LANES, LANES)
                    buf[bs + i, sl] = buf[bs + i, sl] * wv
            return 0

        lax.fori_loop(0, CHUNK // LANES, edge_group, 0)

    stage(0, 0, pltpu.sync_copy)
    for t in range(nbuf - 1):
        start(0, t, bufs[t], gsems[t])

    def block(b, _):
        pb = b % 2
        pb_next = (b + 1) % 2
        has_next = b < nblk - 1

        @pl.when(has_next)
        def _():
            stage(b + 1, pb_next, lambda s, d: pltpu.async_copy(s, d, esem))

        for t in range(bst):
            bi = t % nbuf
            pi = (t + nbuf - 1) % nbuf
            finish(pb, t, bufs[bi], gsems[bi])
            scale(pb, t, bufs[bi])
            if t == 0:
                @pl.when(b > 0)
                def _():
                    scat_wait(pb, t, bufs[pi], ssems[pi])
            else:
                scat_wait(pb, t, bufs[pi], ssems[pi])
            scat_start(pb, t, bufs[bi], ssems[bi])
            nt = t + nbuf - 1
            if nt < bst:
                start(pb, nt, bufs[nt % nbuf], gsems[nt % nbuf])
            else:
                if nt == bst:
                    @pl.when(has_next)
                    def _():
                        stage(b + 1, pb_next,
                              lambda s, d: pltpu.make_async_copy(s, d, esem).wait())

                @pl.when(has_next)
                def _():
                    start(pb_next, nt - bst, bufs[nt % nbuf], gsems[nt % nbuf])
        return 0

    lax.fori_loop(0, nblk, block, 0)
    scat_wait(0, bst - 1, bufs[(bst - 1) % nbuf], ssems[(bst - 1) % nbuf])


def _drain(acc, dbuf, out_hbm, row0):
    del dbuf
    pltpu.sync_copy(acc.at[pl.ds(row0, ROWS_PER_TILE)],
                    out_hbm.at[pl.ds(row0, ROWS_PER_TILE)])


def _spmm_scratch(n, nch, ncols, bst):
    return [
        pltpu.VMEM((2, bst, CHUNK), jnp.int32),
        pltpu.VMEM((2, bst, CHUNK), jnp.int32),
        pltpu.VMEM((2, bst, CHUNK), jnp.float32),
        pltpu.VMEM((CHUNK, ncols), jnp.float32),
        pltpu.VMEM((CHUNK, ncols), jnp.float32),
        pltpu.VMEM((CHUNK, ncols), jnp.float32),
        pltpu.VMEM((CHUNK, ncols), jnp.float32),
        pltpu.VMEM_SHARED((n, ncols), jnp.float32),
    ] + [pltpu.SemaphoreType.DMA] * 9


def _make_spmm1(n, nch, ncols, bst):
    mesh = plsc.VectorSubcoreMesh(core_axis_name="c", subcore_axis_name="s")

    @functools.partial(
        pl.kernel,
        mesh=mesh,
        out_type=[
            jax.ShapeDtypeStruct((n, ncols), jnp.float32),
            jax.ShapeDtypeStruct((n, ncols), jnp.float32),
        ],
        scratch_types=_spmm_scratch(n, nch, ncols, bst),
    )
    def spmm1(h0, h1, src2d, dst2d, w2d, s0, s1,
              srcbuf, dstbuf, wbuf, r0, r1, r2, r3, acc, *sems):
        c = lax.axis_index("c")
        s = lax.axis_index("s")
        row0 = s * ROWS_PER_TILE

        _zero_rows(r0, DRAIN_CHUNK, ncols)
        for j in range(ROWS_PER_TILE // DRAIN_CHUNK):
            pltpu.sync_copy(r0, acc.at[pl.ds(row0 + j * DRAIN_CHUNK, DRAIN_CHUNK)])
        plsc.subcore_barrier()

        for cidx, h_hbm in enumerate([h0, h1]):
            @pl.when(c == cidx)
            def _():
                _spmm_accumulate(h_hbm, acc, src2d, dst2d, w2d, s * nch,
                                 srcbuf, dstbuf, wbuf, [r0, r1, r2, r3],
                                 sems[0:4], sems[4:8], sems[8], nch, ncols, bst)

        plsc.subcore_barrier()

        for cidx, out_hbm in enumerate([s0, s1]):
            @pl.when(c == cidx)
            def _():
                _drain(acc, r0, out_hbm, row0)

    return spmm1


def _make_spmm2(n, nch, ncols, bst):
    mesh = plsc.VectorSubcoreMesh(core_axis_name="c", subcore_axis_name="s")

    @functools.partial(
        pl.kernel,
        mesh=mesh,
        out_type=[
            jax.ShapeDtypeStruct((n, ncols), jnp.float32),
            jax.ShapeDtypeStruct((n, ncols), jnp.float32),
        ],
        scratch_types=_spmm_scratch(n, nch, ncols, bst),
    )
    def spmm2(h2, src2d, dst2d, w2d, p0, p1,
              srcbuf, dstbuf, wbuf, r0, r1, r2, r3, acc, *sems):
        c = lax.axis_index("c")
        s = lax.axis_index("s")
        wid = c * N_SUBCORES + s
        row0 = s * ROWS_PER_TILE

        _zero_rows(r0, DRAIN_CHUNK, ncols)
        for j in range(ROWS_PER_TILE // DRAIN_CHUNK):
            pltpu.sync_copy(r0, acc.at[pl.ds(row0 + j * DRAIN_CHUNK, DRAIN_CHUNK)])
        plsc.subcore_barrier()

        _spmm_accumulate(h2, acc, src2d, dst2d, w2d, wid * nch,
                         srcbuf, dstbuf, wbuf, [r0, r1, r2, r3],
                         sems[0:4], sems[4:8], sems[8], nch, ncols, bst)

        plsc.subcore_barrier()

        for cidx, out_hbm in enumerate([p0, p1]):
            @pl.when(c == cidx)
            def _():
                _drain(acc, r0, out_hbm, row0)

    return spmm2


def _mm1_body(x_ref, w_ref, o0_ref, o1_ref):
    xb = x_ref[...]
    o0_ref[...] = jnp.dot(xb, w_ref[:, :128], preferred_element_type=jnp.float32)
    o1_ref[...] = jnp.dot(xb, w_ref[:, 128:], preferred_element_type=jnp.float32)


def _mm2_body(s0_ref, s1_ref, b1_ref, w2_ref, o_ref):
    a0 = jnp.maximum(s0_ref[...] + b1_ref[0, :128], 0.0)
    a1 = jnp.maximum(s1_ref[...] + b1_ref[0, 128:], 0.0)
    acc = jnp.dot(a0, w2_ref[:128, :], preferred_element_type=jnp.float32)
    acc += jnp.dot(a1, w2_ref[128:, :], preferred_element_type=jnp.float32)
    o_ref[...] = acc


def _combine_body(p0_ref, p1_ref, b2_ref, o_ref):
    o_ref[...] = p0_ref[...] + p1_ref[...] + b2_ref[0, :]


def _pad_edges(src, dst, w, n_parts, e_total, bst, n, npad):
    blk = CHUNK * bst
    per = -(-e_total // (n_parts * blk)) * blk
    e_pad = per * n_parts
    pad = e_pad - e_total
    pidx = jnp.arange(pad, dtype=jnp.int32)
    src_p = jnp.concatenate([src, pidx % n]).reshape(e_pad // CHUNK, CHUNK)
    dst_p = jnp.concatenate([dst, n + pidx % (npad - n)]).reshape(e_pad // CHUNK, CHUNK)
    w_p = jnp.concatenate([w, jnp.zeros((pad,), w.dtype)]).reshape(e_pad // CHUNK, CHUNK)
    return src_p, dst_p, w_p, per // CHUNK


def kernel(x, edge_index, edge_weight, W1, b1, W2, b2):
    n, d_in = x.shape
    npad = N_PAD
    e = edge_weight.shape[0]
    d_h = W1.shape[1]
    d_out = W2.shape[1]
    half = d_h // 2
    bst1, bst2 = 8, 8

    dst = edge_index[0].astype(jnp.int32)
    src = edge_index[1].astype(jnp.int32)
    w = edge_weight

    src1, dst1, w1, nch1 = _pad_edges(src, dst, w, N_SUBCORES, e, bst1, n, npad)
    src2, dst2, w2, nch2 = _pad_edges(src, dst, w, N_SUBCORES * N_CORES, e, bst2, n, npad)

    grid = (n // MM_BLOCK,)
    h0, h1 = pl.pallas_call(
        _mm1_body,
        grid=grid,
        in_specs=[
            pl.BlockSpec((MM_BLOCK, d_in), lambda i: (i, 0)),
            pl.BlockSpec((d_in, d_h), lambda i: (0, 0)),
        ],
        out_specs=[
            pl.BlockSpec((MM_BLOCK, half), lambda i: (i, 0)),
            pl.BlockSpec((MM_BLOCK, half), lambda i: (i, 0)),
        ],
        out_shape=[
            jax.ShapeDtypeStruct((n, half), jnp.float32),
            jax.ShapeDtypeStruct((n, half), jnp.float32),
        ],
    )(x, W1)

    s0, s1 = _make_spmm1(npad, nch1, half, bst1)(h0, h1, src1, dst1, w1)

    h2 = pl.pallas_call(
        _mm2_body,
        grid=grid,
        in_specs=[
            pl.BlockSpec((MM_BLOCK, half), lambda i: (i, 0)),
            pl.BlockSpec((MM_BLOCK, half), lambda i: (i, 0)),
            pl.BlockSpec((1, d_h), lambda i: (0, 0)),
            pl.BlockSpec((d_h, d_out), lambda i: (0, 0)),
        ],
        out_specs=pl.BlockSpec((MM_BLOCK, d_out), lambda i: (i, 0)),
        out_shape=jax.ShapeDtypeStruct((n, d_out), jnp.float32),
    )(s0, s1, b1.reshape(1, d_h), W2)

    p0, p1 = _make_spmm2(npad, nch2, d_out, bst2)(h2, src2, dst2, w2)

    out = pl.pallas_call(
        _combine_body,
        grid=grid,
        in_specs=[
            pl.BlockSpec((MM_BLOCK, d_out), lambda i: (i, 0)),
            pl.BlockSpec((MM_BLOCK, d_out), lambda i: (i, 0)),
            pl.BlockSpec((1, d_out), lambda i: (0, 0)),
        ],
        out_specs=pl.BlockSpec((MM_BLOCK, d_out), lambda i: (i, 0)),
        out_shape=jax.ShapeDtypeStruct((n, d_out), jnp.float32),
    )(p0, p1, b2.reshape(1, d_out))

    return out
